# baseline (speedup 1.0000x reference)
import functools

import jax
import jax.numpy as jnp
from jax import lax
from jax.experimental import pallas as pl
from jax.experimental.pallas import tpu as pltpu
from jax.experimental.pallas import tpu_sc as plsc

_NUM_CORES = 2
_NUM_SUBCORES = 16
_NUM_WORKERS = _NUM_CORES * _NUM_SUBCORES


@functools.lru_cache(maxsize=None)
def _make_copy(seq_len: int, hidden: int, rows_per_w: int):
    mesh = plsc.VectorSubcoreMesh(core_axis_name="c", subcore_axis_name="s")

    @functools.partial(
        pl.kernel,
        mesh=mesh,
        out_type=jax.ShapeDtypeStruct((seq_len, hidden), jnp.float32),
    )
    def copy_kernel(table_hbm, out_hbm):
        wid = lax.axis_index("s") * _NUM_CORES + lax.axis_index("c")
        base = wid * rows_per_w
        pltpu.sync_copy(
            table_hbm.at[pl.ds(base, rows_per_w)],
            out_hbm.at[pl.ds(base, rows_per_w)],
        )

    return copy_kernel


def kernel(x, emb_table):
    seq_len = x.shape[1]
    hidden = emb_table.shape[1]
    assert seq_len % _NUM_WORKERS == 0
    rows_per_w = seq_len // _NUM_WORKERS
    out = _make_copy(seq_len, hidden, rows_per_w)(emb_table)
    return out[None]


# speedup vs baseline: 24.2043x; 24.2043x over previous
import functools

import jax
import jax.numpy as jnp
from jax import lax
from jax.experimental import pallas as pl
from jax.experimental.pallas import tpu as pltpu
from jax.experimental.pallas import tpu_sc as plsc

_NUM_CORES = 2
_NUM_SUBCORES = 16
_NUM_WORKERS = _NUM_CORES * _NUM_SUBCORES
_CHUNK_ROWS = 16


@functools.lru_cache(maxsize=None)
def _make_copy(seq_len: int, hidden: int):
    rows_per_w = seq_len // _NUM_WORKERS
    n_chunks = rows_per_w // _CHUNK_ROWS
    mesh = plsc.VectorSubcoreMesh(core_axis_name="c", subcore_axis_name="s")

    @functools.partial(
        pl.kernel,
        mesh=mesh,
        out_type=jax.ShapeDtypeStruct((seq_len, hidden), jnp.float32),
        scratch_types=[
            pltpu.VMEM((_CHUNK_ROWS, hidden), jnp.float32),
            pltpu.VMEM((_CHUNK_ROWS, hidden), jnp.float32),
            pltpu.SemaphoreType.DMA,
            pltpu.SemaphoreType.DMA,
            pltpu.SemaphoreType.DMA,
            pltpu.SemaphoreType.DMA,
        ],
    )
    def copy_kernel(table_hbm, out_hbm, buf0, buf1, si0, si1, so0, so1):
        wid = lax.axis_index("s") * _NUM_CORES + lax.axis_index("c")
        base = wid * rows_per_w
        bufs = (buf0, buf1)
        isems = (si0, si1)
        osems = (so0, so1)

        def chunk(i):
            return pl.ds(base + i * _CHUNK_ROWS, _CHUNK_ROWS)

        ins = [None] * n_chunks
        outs = [None] * n_chunks
        ins[0] = pltpu.make_async_copy(table_hbm.at[chunk(0)], bufs[0], isems[0])
        ins[0].start()
        for i in range(n_chunks):
            b = i % 2
            if i + 1 < n_chunks:
                if i >= 1:
                    outs[i - 1].wait()
                ins[i + 1] = pltpu.make_async_copy(
                    table_hbm.at[chunk(i + 1)], bufs[1 - b], isems[1 - b]
                )
                ins[i + 1].start()
            ins[i].wait()
            outs[i] = pltpu.make_async_copy(bufs[b], out_hbm.at[chunk(i)], osems[b])
            outs[i].start()
        outs[n_chunks - 2].wait()
        outs[n_chunks - 1].wait()

    return copy_kernel


def kernel(x, emb_table):
    seq_len = x.shape[1]
    hidden = emb_table.shape[1]
    assert seq_len % (_NUM_WORKERS * _CHUNK_ROWS) == 0
    out = _make_copy(seq_len, hidden)(emb_table)
    return out[None]


# speedup vs baseline: 24.8426x; 1.0264x over previous
import functools

import jax
import jax.numpy as jnp
from jax import lax
from jax.experimental import pallas as pl
from jax.experimental.pallas import tpu as pltpu
from jax.experimental.pallas import tpu_sc as plsc

_NUM_CORES = 2
_NUM_SUBCORES = 16
_NUM_WORKERS = _NUM_CORES * _NUM_SUBCORES
_NBUF = 3
_CHUNK_ROWS = 16


@functools.lru_cache(maxsize=None)
def _make_copy(seq_len: int, hidden: int):
    rows_per_w = seq_len // _NUM_WORKERS
    n = rows_per_w // _CHUNK_ROWS
    mesh = plsc.VectorSubcoreMesh(core_axis_name="c", subcore_axis_name="s")

    scratch = [pltpu.VMEM((_CHUNK_ROWS, hidden), jnp.float32)] * _NBUF
    scratch += [pltpu.SemaphoreType.DMA] * (2 * _NBUF)

    @functools.partial(
        pl.kernel,
        mesh=mesh,
        out_type=jax.ShapeDtypeStruct((1, seq_len, hidden), jnp.float32),
        scratch_types=scratch,
    )
    def copy_kernel(table_hbm, out_hbm, *scr):
        bufs = scr[:_NBUF]
        isems = scr[_NBUF : 2 * _NBUF]
        osems = scr[2 * _NBUF :]
        wid = lax.axis_index("s") * _NUM_CORES + lax.axis_index("c")
        base = wid * rows_per_w

        def sl(i):
            return pl.ds(base + i * _CHUNK_ROWS, _CHUNK_ROWS)

        def start_in(i):
            h = pltpu.make_async_copy(table_hbm.at[sl(i)], bufs[i % _NBUF], isems[i % _NBUF])
            h.start()
            return h

        def start_out(i):
            h = pltpu.make_async_copy(bufs[i % _NBUF], out_hbm.at[0, sl(i)], osems[i % _NBUF])
            h.start()
            return h

        in_h = [None] * n
        out_h = [None] * n
        for j in range(min(_NBUF - 1, n)):
            in_h[j] = start_in(j)
        for i in range(n):
            j = i + _NBUF - 1
            if j < n:
                if j - _NBUF >= 0:
                    out_h[j - _NBUF].wait()
                in_h[j] = start_in(j)
            in_h[i].wait()
            out_h[i] = start_out(i)
        for i in range(max(0, n - _NBUF), n):
            out_h[i].wait()

    return copy_kernel


def kernel(x, emb_table):
    seq_len = x.shape[1]
    hidden = emb_table.shape[1]
    assert seq_len % (_NUM_WORKERS * _CHUNK_ROWS) == 0
    return _make_copy(seq_len, hidden)(emb_table)
